# baseline (speedup 1.0000x reference)
import jax
import jax.numpy as jnp
from jax import lax
from jax.experimental import pallas as pl
from jax.experimental.pallas import tpu as pltpu
from jax.experimental.pallas import tpu_sc as plsc

_BATCH = 16
_MAX_LEN = 2048
_TOTAL = 32768
_NC = 2
_NS = 16
_L = 16
_HALF = _MAX_LEN // 2
_SLICE = _TOTAL // _NS
_SWIN = 2 * _HALF


def _body(values_hbm, rowids_hbm, out_hbm, rid_v, tab_v, win_v, buf_v,
          shared, sem, rsem, osem):
    c = lax.axis_index("c")
    s = lax.axis_index("s")

    praw = s * _SLICE + c * _HALF - (_SWIN - _HALF) // 2
    pbase = pl.multiple_of(
        jnp.minimum(jnp.maximum(praw, 0), _TOTAL - _SWIN), _L)
    rid_copy = pltpu.make_async_copy(
        rowids_hbm.at[pl.ds(s * _SLICE, _SLICE)], rid_v, rsem)
    rid_copy.start()
    spec = pltpu.make_async_copy(
        values_hbm.at[pl.ds(pbase, _SWIN)], win_v, sem)
    spec.start()
    rid_copy.wait()

    t = lax.iota(jnp.int32, _L)
    lo = jnp.zeros((_L,), jnp.int32)
    hi = jnp.full((_L,), _SLICE, jnp.int32)

    def bs_step(_, carry):
        lo, hi = carry
        mid = (lo + hi) // 2
        v = plsc.load_gather(rid_v, [jnp.minimum(mid, _SLICE - 1)])
        active = lo < hi
        less = (v < t) & active
        lo = jnp.where(less, mid + 1, lo)
        hi = jnp.where(active & jnp.logical_not(less), mid, hi)
        return lo, hi

    lo, hi = lax.fori_loop(0, 12, bs_step, (lo, hi))
    buf_v[pl.ds(0, _L)] = lo.astype(jnp.float32)
    pltpu.sync_copy(buf_v.at[pl.ds(0, _L)], shared.at[pl.ds(s * _L, _L)])
    plsc.subcore_barrier()

    pltpu.sync_copy(shared, tab_v)
    starts = jnp.zeros((_L,), jnp.float32)
    for i in range(_NS):
        starts = starts + tab_v[pl.ds(i * _L, _L)]
    starts = starts.astype(jnp.int32)
    start_b = jnp.sum(jnp.where(t == s, starts, 0))
    end_b = jnp.where(
        s == _NS - 1,
        jnp.int32(_TOTAL),
        jnp.sum(jnp.where(t == s + 1, starts, 0)),
    )

    col0 = start_b + c * _HALF
    spec.wait()
    hit = (col0 >= pbase) & (col0 + _HALF <= pbase + _SWIN)
    base = jnp.minimum((col0 // _L) * _L, _TOTAL - _SWIN)
    base = pl.multiple_of(jnp.maximum(base, 0), _L)

    @pl.when(jnp.logical_not(hit))
    def _():
        pltpu.sync_copy(values_hbm.at[pl.ds(base, _SWIN)], win_v)

    off = col0 - jnp.where(hit, pbase, base)

    rem = (end_b - col0) - t
    lidx0 = off + t

    def copy_step(j, _):
        valid = j * _L < rem
        g = plsc.load_gather(win_v, [jnp.minimum(lidx0 + j * _L, _SWIN - 1)])
        buf_v[pl.ds(j * _L, _L)] = jnp.where(valid, g, 0.0)
        return 0

    lax.fori_loop(0, _HALF // (2 * _L), copy_step, 0, unroll=4)
    out1 = pltpu.make_async_copy(
        buf_v.at[pl.ds(0, _HALF // 2)],
        out_hbm.at[s, pl.ds(c * _HALF, _HALF // 2)], osem)
    out1.start()
    lax.fori_loop(_HALF // (2 * _L), _HALF // _L, copy_step, 0, unroll=4)
    pltpu.sync_copy(
        buf_v.at[pl.ds(_HALF // 2, _HALF // 2)],
        out_hbm.at[s, pl.ds(c * _HALF + _HALF // 2, _HALF // 2)])
    out1.wait()


def kernel(values, row_ids):
    mesh = plsc.VectorSubcoreMesh(
        core_axis_name="c", subcore_axis_name="s",
        num_cores=_NC, num_subcores=_NS)
    fn = pl.kernel(
        _body,
        out_type=jax.ShapeDtypeStruct((_BATCH, _MAX_LEN), jnp.float32),
        mesh=mesh,
        compiler_params=pltpu.CompilerParams(needs_layout_passes=False),
        scratch_types=[
            pltpu.VMEM((_SLICE,), jnp.int32),
            pltpu.VMEM((_NS * _L,), jnp.float32),
            pltpu.VMEM((_SWIN,), jnp.float32),
            pltpu.VMEM((_HALF,), jnp.float32),
            pltpu.VMEM_SHARED((_NS * _L,), jnp.float32),
            pltpu.SemaphoreType.DMA,
            pltpu.SemaphoreType.DMA,
            pltpu.SemaphoreType.DMA,
        ],
    )
    return fn(values, jnp.asarray(row_ids, jnp.int32))


# speedup vs baseline: 1.0236x; 1.0236x over previous
import jax
import jax.numpy as jnp
from jax import lax
from jax.experimental import pallas as pl
from jax.experimental.pallas import tpu as pltpu
from jax.experimental.pallas import tpu_sc as plsc

_BATCH = 16
_MAX_LEN = 2048
_TOTAL = 32768
_NC = 2
_NS = 16
_L = 16
_HALF = _MAX_LEN // 2
_SLICE = _TOTAL // _NS
_SWIN = 2 * _HALF


def _body(values_hbm, rowids_hbm, out_hbm, rid_v, tab_v, win_v, buf_v,
          shared, sem, rsem):
    c = lax.axis_index("c")
    s = lax.axis_index("s")

    praw = s * _SLICE + c * _HALF - (_SWIN - _HALF) // 2
    pbase = pl.multiple_of(
        jnp.minimum(jnp.maximum(praw, 0), _TOTAL - _SWIN), _L)
    rid_copy = pltpu.make_async_copy(
        rowids_hbm.at[pl.ds(s * _SLICE, _SLICE)], rid_v, rsem)
    rid_copy.start()
    spec = pltpu.make_async_copy(
        values_hbm.at[pl.ds(pbase, _SWIN)], win_v, sem)
    spec.start()
    rid_copy.wait()

    t = lax.iota(jnp.int32, _L)
    lo = jnp.zeros((_L,), jnp.int32)
    hi = jnp.full((_L,), _SLICE, jnp.int32)

    def bs_step(_, carry):
        lo, hi = carry
        mid = (lo + hi) // 2
        v = plsc.load_gather(rid_v, [jnp.minimum(mid, _SLICE - 1)])
        active = lo < hi
        less = (v < t) & active
        lo = jnp.where(less, mid + 1, lo)
        hi = jnp.where(active & jnp.logical_not(less), mid, hi)
        return lo, hi

    lo, hi = lax.fori_loop(0, 12, bs_step, (lo, hi))
    buf_v[pl.ds(0, _L)] = lo.astype(jnp.float32)
    pltpu.sync_copy(buf_v.at[pl.ds(0, _L)], shared.at[pl.ds(s * _L, _L)])
    plsc.subcore_barrier()

    pltpu.sync_copy(shared, tab_v)
    starts = jnp.zeros((_L,), jnp.float32)
    for i in range(_NS):
        starts = starts + tab_v[pl.ds(i * _L, _L)]
    starts = starts.astype(jnp.int32)
    start_b = jnp.sum(jnp.where(t == s, starts, 0))
    end_b = jnp.where(
        s == _NS - 1,
        jnp.int32(_TOTAL),
        jnp.sum(jnp.where(t == s + 1, starts, 0)),
    )

    col0 = start_b + c * _HALF
    spec.wait()
    hit = (col0 >= pbase) & (col0 + _HALF <= pbase + _SWIN)
    base = jnp.minimum((col0 // _L) * _L, _TOTAL - _SWIN)
    base = pl.multiple_of(jnp.maximum(base, 0), _L)

    @pl.when(jnp.logical_not(hit))
    def _():
        pltpu.sync_copy(values_hbm.at[pl.ds(base, _SWIN)], win_v)

    off = col0 - jnp.where(hit, pbase, base)

    rem = (end_b - col0) - t
    lidx0 = off + t

    def copy_step(j, _):
        valid = j * _L < rem
        g = plsc.load_gather(win_v, [jnp.minimum(lidx0 + j * _L, _SWIN - 1)])
        buf_v[pl.ds(j * _L, _L)] = jnp.where(valid, g, 0.0)
        return 0

    lax.fori_loop(0, _HALF // _L, copy_step, 0, unroll=4)

    pltpu.sync_copy(buf_v, out_hbm.at[s, pl.ds(c * _HALF, _HALF)])


def kernel(values, row_ids):
    mesh = plsc.VectorSubcoreMesh(
        core_axis_name="c", subcore_axis_name="s",
        num_cores=_NC, num_subcores=_NS)
    fn = pl.kernel(
        _body,
        out_type=jax.ShapeDtypeStruct((_BATCH, _MAX_LEN), jnp.float32),
        mesh=mesh,
        compiler_params=pltpu.CompilerParams(needs_layout_passes=False),
        scratch_types=[
            pltpu.VMEM((_SLICE,), jnp.int32),
            pltpu.VMEM((_NS * _L,), jnp.float32),
            pltpu.VMEM((_SWIN,), jnp.float32),
            pltpu.VMEM((_HALF,), jnp.float32),
            pltpu.VMEM_SHARED((_NS * _L,), jnp.float32),
            pltpu.SemaphoreType.DMA,
            pltpu.SemaphoreType.DMA,
        ],
    )
    return fn(values, jnp.asarray(row_ids, jnp.int32))
